# baseline (speedup 1.0000x reference)
import dataclasses
import functools

import jax
import jax.numpy as jnp
from jax import lax
from jax.experimental import pallas as pl
from jax.experimental.pallas import tpu as pltpu
from jax.experimental.pallas import tpu_sc as plsc

ROWS = 128
COLS = 32768
K = 16
L = 16
NC = 2
NS = 16
G = 16
U = 4


def _merge_max(a, b):
    return jnp.sort(jnp.maximum(a, jnp.flip(b)))


def _merge_min(a, b):
    return jnp.sort(jnp.minimum(a, jnp.flip(b)))


def kernel(x):
    nw = NC * NS
    rows_per_w = ROWS // nw

    mesh = plsc.VectorSubcoreMesh(core_axis_name="c", subcore_axis_name="s")

    cp = pltpu.CompilerParams()
    if "needs_layout_passes" in pltpu.CompilerParams.__dataclass_fields__:
        cp = dataclasses.replace(cp, needs_layout_passes=False)

    @functools.partial(
        pl.kernel,
        out_type=jax.ShapeDtypeStruct((ROWS, 2 * K), jnp.float32),
        mesh=mesh,
        compiler_params=cp,
        scratch_types=[
            pltpu.VMEM((COLS,), jnp.float32),
            pltpu.VMEM((2 * K,), jnp.float32),
            pltpu.SemaphoreType.DMA,
        ],
    )
    def run(x_hbm, out_hbm, row_v, out_v, sem):
        cid = lax.axis_index("c")
        sid = lax.axis_index("s")
        wid = sid * NC + cid

        def merge_group(gbase, tmax, tmin):
            cs = [jnp.sort(row_v[pl.ds(gbase + k * L, L)]) for k in range(G)]
            t = cs
            while len(t) > 1:
                t = [_merge_max(t[2 * i], t[2 * i + 1])
                     for i in range(len(t) // 2)]
            b = cs
            while len(b) > 1:
                b = [_merge_min(b[2 * i], b[2 * i + 1])
                     for i in range(len(b) // 2)]
            tmax = _merge_max(tmax, t[0])
            tmin = _merge_min(tmin, b[0])
            return tmax, tmin, jnp.min(tmax), jnp.max(tmin)

        @pl.loop(0, rows_per_w)
        def _(r):
            row = wid * rows_per_w + r
            pltpu.async_copy(x_hbm.at[row], row_v, sem).wait()

            neg = jnp.full((L,), -jnp.inf, jnp.float32)
            pos = jnp.full((L,), jnp.inf, jnp.float32)

            def body(i, carry):
                base = i * (U * G * L)
                stats = []
                for g in range(U):
                    gbase = base + g * G * L
                    vmax = row_v[pl.ds(gbase, L)]
                    vmin = vmax
                    for k in range(1, G):
                        c = row_v[pl.ds(gbase + k * L, L)]
                        vmax = jnp.maximum(vmax, c)
                        vmin = jnp.minimum(vmin, c)
                    stats.append((jnp.max(vmax), jnp.min(vmin)))
                for g in range(U):
                    gbase = base + g * G * L
                    smax, smin = stats[g]
                    tmax, tmin, thr_top, thr_bot = carry
                    pred = (smax > thr_top) | (smin < thr_bot)
                    carry = lax.cond(
                        pred,
                        lambda a: merge_group(gbase, a[0], a[1]),
                        lambda a: a,
                        carry)
                return carry

            init = (neg, pos, jnp.float32(-jnp.inf), jnp.float32(jnp.inf))
            tmax, tmin, _, _ = lax.fori_loop(
                0, COLS // (L * G * U), body, init)
            out_v[pl.ds(0, K)] = tmin
            out_v[pl.ds(K, K)] = tmax
            pltpu.sync_copy(out_v, out_hbm.at[row])

    return run(x)


# speedup vs baseline: 2.1604x; 2.1604x over previous
import dataclasses
import functools

import jax
import jax.numpy as jnp
from jax import lax
from jax.experimental import pallas as pl
from jax.experimental.pallas import tpu as pltpu
from jax.experimental.pallas import tpu_sc as plsc

ROWS = 128
COLS = 32768
K = 16
L = 16
NC = 2
NS = 16
G = 16
U = 4
NG = COLS // (G * L)


def _merge_max(a, b):
    return jnp.sort(jnp.maximum(a, jnp.flip(b)))


def _merge_min(a, b):
    return jnp.sort(jnp.minimum(a, jnp.flip(b)))


def _kv_merge_max(av, ai, bv, bi):
    bvf, bif = jnp.flip(bv), jnp.flip(bi)
    m = av >= bvf
    return plsc.sort_key_val(jnp.where(m, av, bvf), jnp.where(m, ai, bif))


def _kv_merge_min(av, ai, bv, bi):
    bvf, bif = jnp.flip(bv), jnp.flip(bi)
    m = av <= bvf
    return plsc.sort_key_val(jnp.where(m, av, bvf), jnp.where(m, ai, bif))


def kernel(x):
    nw = NC * NS
    rows_per_w = ROWS // nw

    mesh = plsc.VectorSubcoreMesh(core_axis_name="c", subcore_axis_name="s")

    cp = pltpu.CompilerParams()
    if "needs_layout_passes" in pltpu.CompilerParams.__dataclass_fields__:
        cp = dataclasses.replace(cp, needs_layout_passes=False)

    @functools.partial(
        pl.kernel,
        out_type=jax.ShapeDtypeStruct((ROWS, 2 * K), jnp.float32),
        mesh=mesh,
        compiler_params=cp,
        scratch_types=[
            pltpu.VMEM((COLS,), jnp.float32),
            pltpu.VMEM((COLS,), jnp.float32),
            pltpu.VMEM((2 * K,), jnp.float32),
            pltpu.SemaphoreType.DMA,
            pltpu.SemaphoreType.DMA,
        ],
    )
    def run(x_hbm, out_hbm, row_a, row_b, out_v, sem_a, sem_b):
        cid = lax.axis_index("c")
        sid = lax.axis_index("s")
        wid = sid * NC + cid
        row0 = wid * rows_per_w

        neg = jnp.full((L,), -jnp.inf, jnp.float32)
        pos = jnp.full((L,), jnp.inf, jnp.float32)
        zero_ids = jnp.zeros((L,), jnp.int32)

        lane = lax.iota(jnp.int32, L)

        def tree(cs, merger):
            t = cs
            while len(t) > 1:
                t = [merger(t[2 * i], t[2 * i + 1])
                     for i in range(len(t) // 2)]
            return t[0]

        def compute_row(row, buf):
            def body(i, carry):
                kx = list(carry[0])
                ix = list(carry[1])
                kn = list(carry[2])
                im = list(carry[3])
                for a in range(U):
                    g = i * U + a
                    gbase = g * (G * L)
                    c = buf[pl.ds(gbase, L)]
                    vmax = c
                    vmin = c
                    for k in range(1, G):
                        c = buf[pl.ds(gbase + k * L, L)]
                        vmax = jnp.maximum(vmax, c)
                        vmin = jnp.minimum(vmin, c)
                    cid = jnp.broadcast_to(g * L, (L,)).astype(jnp.int32) + lane
                    sv, si = plsc.sort_key_val(vmax, cid)
                    kx[a], ix[a] = _kv_merge_max(kx[a], ix[a], sv, si)
                    sv, si = plsc.sort_key_val(vmin, cid)
                    kn[a], im[a] = _kv_merge_min(kn[a], im[a], sv, si)
                return tuple(kx), tuple(ix), tuple(kn), tuple(im)

            init = ((neg,) * U, (zero_ids,) * U, (pos,) * U, (zero_ids,) * U)
            kx, ix, kn, im = lax.fori_loop(0, NG // U, body, init)

            av, ai = _kv_merge_max(kx[0], ix[0], kx[1], ix[1])
            bv, bi = _kv_merge_max(kx[2], ix[2], kx[3], ix[3])
            _, itop = _kv_merge_max(av, ai, bv, bi)
            av, ai = _kv_merge_min(kn[0], im[0], kn[1], im[1])
            bv, bi = _kv_merge_min(kn[2], im[2], kn[3], im[3])
            _, ibot = _kv_merge_min(av, ai, bv, bi)

            base_t = (itop >> 4) * (G * L) + (itop & (L - 1))
            base_b = (ibot >> 4) * (G * L) + (ibot & (L - 1))
            cst = [jnp.sort(plsc.load_gather(buf, [base_t + k * L]))
                   for k in range(G)]
            tmax = tree(cst, _merge_max)
            csb = [jnp.sort(plsc.load_gather(buf, [base_b + k * L]))
                   for k in range(G)]
            tmin = tree(csb, _merge_min)
            out_v[pl.ds(0, K)] = tmin
            out_v[pl.ds(K, K)] = tmax
            pltpu.sync_copy(out_v, out_hbm.at[row])

        bufs = (row_a, row_b)
        sems = (sem_a, sem_b)
        copies = [pltpu.async_copy(x_hbm.at[row0], row_a, sem_a)]
        for r in range(rows_per_w):
            if r + 1 < rows_per_w:
                copies.append(pltpu.async_copy(
                    x_hbm.at[row0 + r + 1], bufs[(r + 1) % 2],
                    sems[(r + 1) % 2]))
            copies[r].wait()
            compute_row(row0 + r, bufs[r % 2])

    return run(x)
